# baseline (speedup 1.0000x reference)
import functools

import jax
import jax.numpy as jnp
from jax import lax
from jax.experimental import pallas as pl
from jax.experimental.pallas import tpu as pltpu
from jax.experimental.pallas import tpu_sc as plsc

U = 25000
I = 25000
N = U + I
E = 800000
D = 64
DH = D // 2

C = 80
EDGES_PER_TILE = E // 16
B = 2000
BLOCKS = EDGES_PER_TILE // B
BCHUNKS = B // C
ZROWS = 125
ROWS_PER_TILE = N // 16

_mesh = plsc.VectorSubcoreMesh(core_axis_name="c", subcore_axis_name="s")

_GATHER_DN = lax.GatherDimensionNumbers(
    offset_dims=(), collapsed_slice_dims=(0,), start_index_map=(0,))


def _bcast_lane(v, l):
    idx = jnp.full((16, 1), l, jnp.int32)
    return lax.gather(v, idx, _GATHER_DN, (1,),
                      mode=lax.GatherScatterMode.PROMISE_IN_BOUNDS)


@functools.partial(
    pl.kernel,
    mesh=_mesh,
    out_type=jax.ShapeDtypeStruct((2, N, DH), jnp.float32),
    scratch_types=[
        pltpu.VMEM((B,), jnp.int32),
        pltpu.VMEM((B // C, C), jnp.int32),
        pltpu.VMEM((B,), jnp.float32),
        pltpu.VMEM((6, C, DH), jnp.float32),
        pltpu.VMEM((ZROWS, DH), jnp.float32),
        pltpu.VMEM_SHARED((N, DH), jnp.float32),
    ] + [pltpu.SemaphoreType.DMA] * 12,
    compiler_params=pltpu.CompilerParams(use_tc_tiling_on_sc=False),
)
def _layer(ego, src_h, dst_h, val_h, out, src_i, dstb, val_f, rows, zblk,
           acc, g0, g1, g2, g3, g4, g5, s0, s1, s2, s3, s4, s5):
    cid = lax.axis_index("c")
    sid = lax.axis_index("s")
    gsem = (g0, g1, g2, g3, g4, g5)
    ssem = (s0, s1, s2, s3, s4, s5)
    my_ego = ego.at[cid]
    my_out = out.at[cid]

    zero16 = jnp.zeros((16,), jnp.float32)
    for r in range(ZROWS):
        for j in range(DH // 16):
            zblk[r, pl.ds(j * 16, 16)] = zero16

    def zero_body(r, _):
        off = sid * ROWS_PER_TILE + r * ZROWS
        pltpu.sync_copy(zblk, acc.at[pl.ds(off, ZROWS)])
        return _

    lax.fori_loop(0, ROWS_PER_TILE // ZROWS, zero_body, None)
    plsc.subcore_barrier()

    def issue_gather(g, p):
        goff = pl.multiple_of(g * C, 8)
        pltpu.async_copy(my_ego.at[src_i.at[pl.ds(goff, C)]],
                         rows.at[p], gsem[p])

    def stage(g, p):
        coff = pl.multiple_of(g * C, 8)
        q = (p + 3) % 6

        @pl.when(g + 3 < BCHUNKS)
        def _():
            @pl.when(g >= 3)
            def _():
                pltpu.make_async_copy(rows.at[q], acc.at[dstb.at[g - 3]],
                                      ssem[q]).wait()

            issue_gather(g + 3, q)

        pltpu.make_async_copy(my_ego.at[src_i.at[pl.ds(coff, C)]],
                              rows.at[p], gsem[p]).wait()

        for ve in range(C // 16):
            vbs = val_f[pl.ds(coff + ve * 16, 16)]
            for l in range(16):
                e = ve * 16 + l
                vb = _bcast_lane(vbs, l)
                for j in range(DH // 16):
                    sl = pl.ds(j * 16, 16)
                    rows[p, e, sl] = rows[p, e, sl] * vb

        pltpu.async_copy(rows.at[p], acc.at[dstb.at[g]], ssem[p], add=True)

    def block_body(b, _):
        boff = pl.multiple_of(sid * EDGES_PER_TILE + b * B, 8)
        brow = sid * (EDGES_PER_TILE // C) + b * (B // C)
        pltpu.sync_copy(src_h.at[pl.ds(boff, B)], src_i)
        pltpu.sync_copy(dst_h.at[pl.ds(brow, B // C)], dstb)
        pltpu.sync_copy(val_h.at[pl.ds(boff, B)], val_f)

        issue_gather(0, 0)
        issue_gather(1, 1)
        issue_gather(2, 2)
        stage(0, 0)

        def six_body(k, _):
            for j in range(6):
                stage(1 + 6 * k + j, (1 + j) % 6)
            return _

        lax.fori_loop(0, (BCHUNKS - 1) // 6, six_body, None)
        for i, bb in enumerate((1, 2, 3, 4, 5, 0)):
            pltpu.make_async_copy(rows.at[bb],
                                  acc.at[dstb.at[BCHUNKS - 6 + i]],
                                  ssem[bb]).wait()
        return _

    lax.fori_loop(0, BLOCKS, block_body, None)
    plsc.subcore_barrier()

    off = sid * ROWS_PER_TILE
    pltpu.sync_copy(acc.at[pl.ds(off, ROWS_PER_TILE)],
                    my_out.at[pl.ds(off, ROWS_PER_TILE)])


def _mean_body(a, b, c, d, o):
    for h in range(2):
        o[:, pl.ds(h * DH, DH)] = (
            a[h] + b[h] + c[h] + d[h]) * 0.25


_mean4 = pl.pallas_call(
    _mean_body,
    grid=(50,),
    in_specs=[pl.BlockSpec((2, 1000, DH), lambda i: (0, i, 0))] * 4,
    out_specs=pl.BlockSpec((1000, D), lambda i: (i, 0)),
    out_shape=jax.ShapeDtypeStruct((N, D), jnp.float32),
)


def kernel(edge_index, edge_values, user_emb, item_emb):
    ego0 = jnp.concatenate([user_emb, item_emb], axis=0)
    ego0c = jnp.stack([ego0[:, :DH], ego0[:, DH:]], axis=0)
    src = edge_index[0]
    dst2d = edge_index[1].reshape(E // C, C)
    e1 = _layer(ego0c, src, dst2d, edge_values)
    e2 = _layer(e1, src, dst2d, edge_values)
    e3 = _layer(e2, src, dst2d, edge_values)
    final = _mean4(ego0c, e1, e2, e3)
    return final[:U], final[U:]


# speedup vs baseline: 1.0826x; 1.0826x over previous
import functools

import jax
import jax.numpy as jnp
from jax import lax
from jax.experimental import pallas as pl
from jax.experimental.pallas import tpu as pltpu
from jax.experimental.pallas import tpu_sc as plsc

U = 25000
I = 25000
N = U + I
E = 800000
D = 64
DH = D // 2

C = 80
EDGES_PER_TILE = E // 16
B = 2000
BLOCKS = EDGES_PER_TILE // B
BCHUNKS = B // C
ZROWS = 125
ROWS_PER_TILE = N // 16

_mesh = plsc.VectorSubcoreMesh(core_axis_name="c", subcore_axis_name="s")

_GATHER_DN = lax.GatherDimensionNumbers(
    offset_dims=(), collapsed_slice_dims=(0,), start_index_map=(0,))


def _bcast_lane(v, l):
    idx = jnp.full((16, 1), l, jnp.int32)
    return lax.gather(v, idx, _GATHER_DN, (1,),
                      mode=lax.GatherScatterMode.PROMISE_IN_BOUNDS)


@functools.partial(
    pl.kernel,
    mesh=_mesh,
    out_type=[jax.ShapeDtypeStruct((2, N, DH), jnp.float32)] * 3,
    scratch_types=[
        pltpu.VMEM((B,), jnp.int32),
        pltpu.VMEM((B // C, C), jnp.int32),
        pltpu.VMEM((B,), jnp.float32),
        pltpu.VMEM((8, C, DH), jnp.float32),
        pltpu.VMEM((ZROWS, DH), jnp.float32),
        pltpu.VMEM_SHARED((N, DH), jnp.float32),
    ] + [pltpu.SemaphoreType.DMA] * 16,
    compiler_params=pltpu.CompilerParams(use_tc_tiling_on_sc=False),
)
def _gcn3(ego, src_h, dst_h, val_h, out1, out2, out3, src_i, dstb, val_f,
          rows, zblk, acc, g0, g1, g2, g3, g4, g5, g6, g7,
          s0, s1, s2, s3, s4, s5, s6, s7):
    cid = lax.axis_index("c")
    sid = lax.axis_index("s")
    gsem = (g0, g1, g2, g3, g4, g5, g6, g7)
    ssem = (s0, s1, s2, s3, s4, s5, s6, s7)

    zero16 = jnp.zeros((16,), jnp.float32)
    for r in range(ZROWS):
        for j in range(DH // 16):
            zblk[r, pl.ds(j * 16, 16)] = zero16

    def run_layer(table, dest):
        my_tab = table.at[cid]
        my_out = dest.at[cid]

        def zero_body(r, _):
            off = sid * ROWS_PER_TILE + r * ZROWS
            pltpu.sync_copy(zblk, acc.at[pl.ds(off, ZROWS)])
            return _

        lax.fori_loop(0, ROWS_PER_TILE // ZROWS, zero_body, None)
        plsc.subcore_barrier()

        def issue_gather(g, p):
            goff = pl.multiple_of(g * C, 8)
            pltpu.async_copy(my_tab.at[src_i.at[pl.ds(goff, C)]],
                             rows.at[p], gsem[p])

        def stage(g, p):
            coff = pl.multiple_of(g * C, 8)
            q = (p + 4) % 8

            @pl.when(g + 4 < BCHUNKS)
            def _():
                @pl.when(g >= 4)
                def _():
                    pltpu.make_async_copy(rows.at[q],
                                          acc.at[dstb.at[g - 4]],
                                          ssem[q]).wait()

                issue_gather(g + 4, q)

            pltpu.make_async_copy(my_tab.at[src_i.at[pl.ds(coff, C)]],
                                  rows.at[p], gsem[p]).wait()

            def scale_body(ve, _):
                vbs = val_f[pl.ds(coff + ve * 16, 16)]
                for l in range(16):
                    for j in range(DH // 16):
                        sl = pl.ds(j * 16, 16)
                        rows[p, ve * 16 + l, sl] = (
                            rows[p, ve * 16 + l, sl] * _bcast_lane(vbs, l))
                return _

            lax.fori_loop(0, C // 16, scale_body, None)

            pltpu.async_copy(rows.at[p], acc.at[dstb.at[g]], ssem[p],
                             add=True)

        def block_body(b, _):
            boff = pl.multiple_of(sid * EDGES_PER_TILE + b * B, 8)
            brow = sid * (EDGES_PER_TILE // C) + b * (B // C)
            pltpu.sync_copy(src_h.at[pl.ds(boff, B)], src_i)
            pltpu.sync_copy(dst_h.at[pl.ds(brow, B // C)], dstb)
            pltpu.sync_copy(val_h.at[pl.ds(boff, B)], val_f)

            issue_gather(0, 0)
            issue_gather(1, 1)
            issue_gather(2, 2)
            issue_gather(3, 3)
            stage(0, 0)

            def eight_body(k, _):
                for j in range(8):
                    stage(1 + 8 * k + j, (1 + j) % 8)
                return _

            lax.fori_loop(0, (BCHUNKS - 1) // 8, eight_body, None)
            for i, bb in enumerate((1, 2, 3, 4, 5, 6, 7, 0)):
                pltpu.make_async_copy(
                    rows.at[bb], acc.at[dstb.at[BCHUNKS - 8 + i]],
                    ssem[bb]).wait()
            return _

        lax.fori_loop(0, BLOCKS, block_body, None)
        plsc.subcore_barrier()

        off = sid * ROWS_PER_TILE
        pltpu.sync_copy(acc.at[pl.ds(off, ROWS_PER_TILE)],
                        my_out.at[pl.ds(off, ROWS_PER_TILE)])
        plsc.subcore_barrier()

    run_layer(ego, out1)
    run_layer(out1, out2)
    run_layer(out2, out3)


def _mean_body(a, b, c, d, o):
    for h in range(2):
        o[:, pl.ds(h * DH, DH)] = (
            a[h] + b[h] + c[h] + d[h]) * 0.25


_mean4 = pl.pallas_call(
    _mean_body,
    grid=(50,),
    in_specs=[pl.BlockSpec((2, 1000, DH), lambda i: (0, i, 0))] * 4,
    out_specs=pl.BlockSpec((1000, D), lambda i: (i, 0)),
    out_shape=jax.ShapeDtypeStruct((N, D), jnp.float32),
)


def kernel(edge_index, edge_values, user_emb, item_emb):
    ego0 = jnp.concatenate([user_emb, item_emb], axis=0)
    ego0c = jnp.stack([ego0[:, :DH], ego0[:, DH:]], axis=0)
    src = edge_index[0]
    dst2d = edge_index[1].reshape(E // C, C)
    e1, e2, e3 = _gcn3(ego0c, src, dst2d, edge_values)
    final = _mean4(ego0c, e1, e2, e3)
    return final[:U], final[U:]


# speedup vs baseline: 1.2103x; 1.1180x over previous
import functools

import jax
import jax.numpy as jnp
from jax import lax
from jax.experimental import pallas as pl
from jax.experimental.pallas import tpu as pltpu
from jax.experimental.pallas import tpu_sc as plsc

U = 25000
I = 25000
N = U + I
E = 800000
D = 64
DH = D // 2

C = 80
EDGES_PER_TILE = E // 16
B = 2000
BLOCKS = EDGES_PER_TILE // B
BCHUNKS = B // C
ROWS_PER_TILE = N // 16

_mesh = plsc.VectorSubcoreMesh(core_axis_name="c", subcore_axis_name="s")

_GATHER_DN = lax.GatherDimensionNumbers(
    offset_dims=(), collapsed_slice_dims=(0,), start_index_map=(0,))


def _bcast_lane(v, l):
    idx = jnp.full((16, 1), l, jnp.int32)
    return lax.gather(v, idx, _GATHER_DN, (1,),
                      mode=lax.GatherScatterMode.PROMISE_IN_BOUNDS)


@functools.partial(
    pl.kernel,
    mesh=_mesh,
    out_type=jax.ShapeDtypeStruct((2, N, DH), jnp.float32),
    scratch_types=[
        pltpu.VMEM((2, B), jnp.int32),
        pltpu.VMEM((2, B // C, C), jnp.int32),
        pltpu.VMEM((B,), jnp.float32),
        pltpu.VMEM((8, C, DH), jnp.float32),
        pltpu.VMEM_SHARED((N, DH), jnp.float32),
    ] + [pltpu.SemaphoreType.DMA] * 18,
    compiler_params=pltpu.CompilerParams(use_tc_tiling_on_sc=False),
)
def _layer(ego, src_h, dst_h, val_h, out, src_i, dstb, val_f,
           rows, acc, g0, g1, g2, g3, g4, g5, g6, g7,
           s0, s1, s2, s3, s4, s5, s6, s7, i0, i1):
    cid = lax.axis_index("c")
    sid = lax.axis_index("s")
    gsem = (g0, g1, g2, g3, g4, g5, g6, g7)
    ssem = (s0, s1, s2, s3, s4, s5, s6, s7)
    isem = (i0, i1)

    zero16 = jnp.zeros((16,), jnp.float32)
    for r in range(C):
        for j in range(DH // 16):
            rows[0, r, pl.ds(j * 16, 16)] = zero16

    if True:
        my_tab = ego.at[cid]
        my_out = out.at[cid]

        def zero_body(r, _):
            off = sid * ROWS_PER_TILE + r * C
            pltpu.sync_copy(rows.at[0], acc.at[pl.ds(off, C)])
            return _

        lax.fori_loop(0, ROWS_PER_TILE // C, zero_body, None)
        tail = ROWS_PER_TILE - (ROWS_PER_TILE // C) * C
        toff = sid * ROWS_PER_TILE + (ROWS_PER_TILE // C) * C
        pltpu.sync_copy(rows.at[0].at[pl.ds(0, tail)],
                        acc.at[pl.ds(toff, tail)])
        plsc.subcore_barrier()

        def index_copies(b, par):
            boff = pl.multiple_of(sid * EDGES_PER_TILE + b * B, 8)
            brow = sid * (EDGES_PER_TILE // C) + b * (B // C)
            return (
                pltpu.make_async_copy(src_h.at[pl.ds(boff, B)],
                                      src_i.at[par], isem[par]),
                pltpu.make_async_copy(dst_h.at[pl.ds(brow, B // C)],
                                      dstb.at[par], isem[par]),
            )

        def issue_gather(g, p, par):
            goff = pl.multiple_of(g * C, 8)
            pltpu.async_copy(my_tab.at[src_i.at[par].at[pl.ds(goff, C)]],
                             rows.at[p], gsem[p])

        def stage(g, p, par):
            coff = pl.multiple_of(g * C, 8)
            q = (p + 4) % 8

            @pl.when(g + 4 < BCHUNKS)
            def _():
                @pl.when(g >= 4)
                def _():
                    pltpu.make_async_copy(rows.at[q],
                                          acc.at[dstb.at[par].at[g - 4]],
                                          ssem[q]).wait()

                issue_gather(g + 4, q, par)

            pltpu.make_async_copy(
                my_tab.at[src_i.at[par].at[pl.ds(coff, C)]],
                rows.at[p], gsem[p]).wait()

            def scale_body(ve, _):
                vbs = val_f[pl.ds(coff + ve * 16, 16)]
                for l in range(16):
                    for j in range(DH // 16):
                        sl = pl.ds(j * 16, 16)
                        rows[p, ve * 16 + l, sl] = (
                            rows[p, ve * 16 + l, sl] * _bcast_lane(vbs, l))
                return _

            lax.fori_loop(0, C // 16, scale_body, None)

            pltpu.async_copy(rows.at[p], acc.at[dstb.at[par].at[g]],
                             ssem[p], add=True)

        def block_body(b, par):
            for d in index_copies(b, par):
                d.wait()
            boff = pl.multiple_of(sid * EDGES_PER_TILE + b * B, 8)
            pltpu.sync_copy(val_h.at[pl.ds(boff, B)], val_f)

            @pl.when(b + 1 < BLOCKS)
            def _():
                for d in index_copies(b + 1, 1 - par):
                    d.start()

            issue_gather(0, 0, par)
            issue_gather(1, 1, par)
            issue_gather(2, 2, par)
            issue_gather(3, 3, par)
            stage(0, 0, par)

            def eight_body(k, _):
                for j in range(8):
                    stage(1 + 8 * k + j, (1 + j) % 8, par)
                return _

            lax.fori_loop(0, (BCHUNKS - 1) // 8, eight_body, None)
            for i, bb in enumerate((1, 2, 3, 4, 5, 6, 7, 0)):
                pltpu.make_async_copy(
                    rows.at[bb], acc.at[dstb.at[par].at[BCHUNKS - 8 + i]],
                    ssem[bb]).wait()

        for d in index_copies(0, 0):
            d.start()
        block_body(0, 0)

        def pair_body(k, _):
            block_body(2 * k + 1, 1)
            block_body(2 * k + 2, 0)
            return _

        lax.fori_loop(0, (BLOCKS - 1) // 2, pair_body, None)
        plsc.subcore_barrier()

        off = sid * ROWS_PER_TILE
        pltpu.sync_copy(acc.at[pl.ds(off, ROWS_PER_TILE)],
                        my_out.at[pl.ds(off, ROWS_PER_TILE)])


def _mean_body(a, b, c, d, o):
    for h in range(2):
        o[:, pl.ds(h * DH, DH)] = (
            a[h] + b[h] + c[h] + d[h]) * 0.25


_mean4 = pl.pallas_call(
    _mean_body,
    grid=(50,),
    in_specs=[pl.BlockSpec((2, 1000, DH), lambda i: (0, i, 0))] * 4,
    out_specs=pl.BlockSpec((1000, D), lambda i: (i, 0)),
    out_shape=jax.ShapeDtypeStruct((N, D), jnp.float32),
)


def kernel(edge_index, edge_values, user_emb, item_emb):
    ego0 = jnp.concatenate([user_emb, item_emb], axis=0)
    ego0c = jnp.stack([ego0[:, :DH], ego0[:, DH:]], axis=0)
    src = edge_index[0]
    dst2d = edge_index[1].reshape(E // C, C)
    e1 = _layer(ego0c, src, dst2d, edge_values)
    e2 = _layer(e1, src, dst2d, edge_values)
    e3 = _layer(e2, src, dst2d, edge_values)
    final = _mean4(ego0c, e1, e2, e3)
    return final[:U], final[U:]


# speedup vs baseline: 1.2106x; 1.0003x over previous
import functools

import jax
import jax.numpy as jnp
from jax import lax
from jax.experimental import pallas as pl
from jax.experimental.pallas import tpu as pltpu
from jax.experimental.pallas import tpu_sc as plsc

U = 25000
I = 25000
N = U + I
E = 800000
D = 64
DH = D // 2

C = 80
EDGES_PER_TILE = E // 16
B = 2000
BLOCKS = EDGES_PER_TILE // B
BCHUNKS = B // C
ROWS_PER_TILE = N // 16

_mesh = plsc.VectorSubcoreMesh(core_axis_name="c", subcore_axis_name="s")

_GATHER_DN = lax.GatherDimensionNumbers(
    offset_dims=(), collapsed_slice_dims=(0,), start_index_map=(0,))


def _bcast_lane(v, l):
    idx = jnp.full((16, 1), l, jnp.int32)
    return lax.gather(v, idx, _GATHER_DN, (1,),
                      mode=lax.GatherScatterMode.PROMISE_IN_BOUNDS)


@functools.partial(
    pl.kernel,
    mesh=_mesh,
    out_type=jax.ShapeDtypeStruct((2, N, DH), jnp.float32),
    scratch_types=[
        pltpu.VMEM((2, B), jnp.int32),
        pltpu.VMEM((2, B // C, C), jnp.int32),
        pltpu.VMEM((B,), jnp.float32),
        pltpu.VMEM((8, C, DH), jnp.float32),
        pltpu.VMEM_SHARED((N, DH), jnp.float32),
    ] + [pltpu.SemaphoreType.DMA] * 18,
    compiler_params=pltpu.CompilerParams(use_tc_tiling_on_sc=False),
)
def _layer(ego, src_h, dst_h, val_h, out, src_i, dstb, val_f,
           rows, acc, g0, g1, g2, g3, g4, g5, g6, g7,
           s0, s1, s2, s3, s4, s5, s6, s7, i0, i1):
    cid = lax.axis_index("c")
    sid = lax.axis_index("s")
    gsem = (g0, g1, g2, g3, g4, g5, g6, g7)
    ssem = (s0, s1, s2, s3, s4, s5, s6, s7)
    isem = (i0, i1)

    zero16 = jnp.zeros((16,), jnp.float32)
    for r in range(C):
        for j in range(DH // 16):
            rows[0, r, pl.ds(j * 16, 16)] = zero16

    my_tab = ego.at[cid]
    my_out = out.at[cid]

    def zero_body(r, _):
        off = sid * ROWS_PER_TILE + r * C
        pltpu.sync_copy(rows.at[0], acc.at[pl.ds(off, C)])
        return _

    lax.fori_loop(0, ROWS_PER_TILE // C, zero_body, None)
    tail = ROWS_PER_TILE - (ROWS_PER_TILE // C) * C
    toff = sid * ROWS_PER_TILE + (ROWS_PER_TILE // C) * C
    pltpu.sync_copy(rows.at[0].at[pl.ds(0, tail)],
                    acc.at[pl.ds(toff, tail)])
    plsc.subcore_barrier()

    def index_copies(b, par):
        boff = pl.multiple_of(sid * EDGES_PER_TILE + b * B, 8)
        brow = sid * (EDGES_PER_TILE // C) + b * (B // C)
        return (
            pltpu.make_async_copy(src_h.at[pl.ds(boff, B)],
                                  src_i.at[par], isem[par]),
            pltpu.make_async_copy(dst_h.at[pl.ds(brow, B // C)],
                                  dstb.at[par], isem[par]),
        )

    def issue_gather(g, p, par):
        goff = pl.multiple_of(g * C, 8)
        pltpu.async_copy(my_tab.at[src_i.at[par].at[pl.ds(goff, C)]],
                         rows.at[p], gsem[p])

    def stage(g, p, par):
        coff = pl.multiple_of(g * C, 8)
        q = (p + 4) % 8

        @pl.when(g + 4 < BCHUNKS)
        def _():
            @pl.when(g >= 4)
            def _():
                pltpu.make_async_copy(rows.at[q],
                                      acc.at[dstb.at[par].at[g - 4]],
                                      ssem[q]).wait()

            issue_gather(g + 4, q, par)

        pltpu.make_async_copy(
            my_tab.at[src_i.at[par].at[pl.ds(coff, C)]],
            rows.at[p], gsem[p]).wait()

        def scale_body(ve, _):
            vbs = val_f[pl.ds(coff + ve * 16, 16)]
            for l in range(16):
                for j in range(DH // 16):
                    sl = pl.ds(j * 16, 16)
                    rows[p, ve * 16 + l, sl] = (
                        rows[p, ve * 16 + l, sl] * _bcast_lane(vbs, l))
            return _

        lax.fori_loop(0, C // 16, scale_body, None)

        pltpu.async_copy(rows.at[p], acc.at[dstb.at[par].at[g]],
                         ssem[p], add=True)

    def block_body(b, par):
        for d in index_copies(b, par):
            d.wait()
        boff = pl.multiple_of(sid * EDGES_PER_TILE + b * B, 8)
        pltpu.sync_copy(val_h.at[pl.ds(boff, B)], val_f)

        @pl.when(b + 1 < BLOCKS)
        def _():
            for d in index_copies(b + 1, 1 - par):
                d.start()

        issue_gather(0, 0, par)
        issue_gather(1, 1, par)
        issue_gather(2, 2, par)
        issue_gather(3, 3, par)
        stage(0, 0, par)

        def eight_body(k, _):
            for j in range(8):
                stage(1 + 8 * k + j, (1 + j) % 8, par)
            return _

        lax.fori_loop(0, (BCHUNKS - 1) // 8, eight_body, None)
        for i, bb in enumerate((1, 2, 3, 4, 5, 6, 7, 0)):
            pltpu.make_async_copy(
                rows.at[bb], acc.at[dstb.at[par].at[BCHUNKS - 8 + i]],
                ssem[bb]).wait()

    for d in index_copies(0, 0):
        d.start()
    block_body(0, 0)

    def pair_body(k, _):
        block_body(2 * k + 1, 1)
        block_body(2 * k + 2, 0)
        return _

    lax.fori_loop(0, (BLOCKS - 1) // 2, pair_body, None)
    plsc.subcore_barrier()

    off = sid * ROWS_PER_TILE
    pltpu.sync_copy(acc.at[pl.ds(off, ROWS_PER_TILE)],
                    my_out.at[pl.ds(off, ROWS_PER_TILE)])


def _mean_body(a, b, c, d, o):
    for h in range(2):
        o[:, pl.ds(h * DH, DH)] = (
            a[h] + b[h] + c[h] + d[h]) * 0.25


_mean4 = pl.pallas_call(
    _mean_body,
    grid=(50,),
    in_specs=[pl.BlockSpec((2, 1000, DH), lambda i: (0, i, 0))] * 4,
    out_specs=pl.BlockSpec((1000, D), lambda i: (i, 0)),
    out_shape=jax.ShapeDtypeStruct((N, D), jnp.float32),
)


def kernel(edge_index, edge_values, user_emb, item_emb):
    ego0 = jnp.concatenate([user_emb, item_emb], axis=0)
    ego0c = jnp.stack([ego0[:, :DH], ego0[:, DH:]], axis=0)
    src = edge_index[0]
    dst2d = edge_index[1].reshape(E // C, C)
    e1 = _layer(ego0c, src, dst2d, edge_values)
    e2 = _layer(e1, src, dst2d, edge_values)
    e3 = _layer(e2, src, dst2d, edge_values)
    final = _mean4(ego0c, e1, e2, e3)
    return final[:U], final[U:]
